# Initial kernel scaffold; baseline (speedup 1.0000x reference)
import jax
import jax.numpy as jnp
from jax.experimental import pallas as pl


def kernel(x, kernel, gamma, beta, mov_mean, mov_var):
    raise NotImplementedError("write your pallas kernel here")


# speedup vs baseline: 1.9184x; 1.9184x over previous
import jax
import jax.numpy as jnp
from jax.experimental import pallas as pl
from jax.experimental.pallas import tpu as pltpu

_DILATIONS = (1, 2, 4, 8, 16)
_ND = len(_DILATIONS)
_BN_EPS = 1e-3
_HALO = 16
_HB = 32
_W = 256
_W2 = _W // 2
_W2P = (_W + 2 * _HALO) // 2

_INV_SQRT2 = 0.7071067811865476


def _window(pt_ref, r0, s):
    if s % 2 == 0:
        c0 = (_HALO + s) // 2
        return pt_ref[r0:r0 + _HB, c0:c0 + _W2, :]
    p0 = (_HALO + s - 1) // 2
    return jnp.concatenate(
        [pt_ref[r0:r0 + _HB, p0:p0 + _W2, 64:128],
         pt_ref[r0:r0 + _HB, p0 + 1:p0 + 1 + _W2, 0:64]], axis=-1)


def _body(xa_ref, xb_ref, kr_ref, sc_ref, sh_ref,
          o0, o1, o2, o3, o4, pt_ref):
    outs = (o0, o1, o2, o3, o4)
    xa = xa_ref[0].reshape(_HB * _W2P, 128)
    xb = xb_ref[0].reshape(_HB * _W2P, 128)
    for t in range(9):
        kt = kr_ref[t]
        pa = jnp.dot(xa, kt, preferred_element_type=jnp.float32)
        pb = jnp.dot(xb, kt, preferred_element_type=jnp.float32)
        pt_ref[0:_HB] = pa.reshape(_HB, _W2P, 128)
        pt_ref[_HB:2 * _HB] = pb.reshape(_HB, _W2P, 128)
        ky, kx = divmod(t, 3)
        for di, d in enumerate(_DILATIONS):
            r0 = _HALO + d * (ky - 1)
            win = _window(pt_ref, r0, d * (kx - 1))[None]
            if t == 0:
                outs[di][...] = win
            else:
                outs[di][...] += win
    for di in range(_ND):
        for r in range(0, _HB, 8):
            y = outs[di][0, r:r + 8] * sc_ref[di] + sh_ref[di]
            outs[di][0, r:r + 8] = 0.5 * y * (1.0 + jax.lax.erf(y * _INV_SQRT2))


def kernel(x, kernel, gamma, beta, mov_mean, mov_var):
    B, H, W, C = x.shape
    scale = gamma * jax.lax.rsqrt(mov_var + _BN_EPS)
    shift = beta - mov_mean * scale
    sc2 = jnp.concatenate([scale, scale], axis=-1)
    sh2 = jnp.concatenate([shift, shift], axis=-1)
    xp = jnp.pad(x, ((0, 0), (_HALO, _HALO), (_HALO, _HALO), (0, 0)))
    xp = xp.astype(jnp.bfloat16).reshape(B, H + 2 * _HALO, _W2P, 2 * C)
    kb = kernel.reshape(9, C, C).astype(jnp.bfloat16)
    z = jnp.zeros((9, C, C), jnp.bfloat16)
    kr = jnp.concatenate(
        [jnp.concatenate([kb, z], -1), jnp.concatenate([z, kb], -1)], axis=1)

    nh = H // _HB
    grid = (B, nh)
    blk_in = (1, _HB, _W2P, 2 * C)
    out_sds = jax.ShapeDtypeStruct((B, H, _W2, 2 * C), jnp.float32)
    out_spec = pl.BlockSpec((1, _HB, _W2, 2 * C), lambda b, i: (b, i, 0, 0))

    outs = pl.pallas_call(
        _body,
        grid=grid,
        in_specs=[
            pl.BlockSpec(blk_in, lambda b, i: (b, i, 0, 0)),
            pl.BlockSpec(blk_in, lambda b, i: (b, i + 1, 0, 0)),
            pl.BlockSpec((9, 2 * C, 2 * C), lambda b, i: (0, 0, 0)),
            pl.BlockSpec((_ND, 2 * C), lambda b, i: (0, 0)),
            pl.BlockSpec((_ND, 2 * C), lambda b, i: (0, 0)),
        ],
        out_specs=[out_spec] * _ND,
        out_shape=[out_sds] * _ND,
        scratch_shapes=[pltpu.VMEM((2 * _HB, _W2P, 2 * C), jnp.float32)],
        compiler_params=pltpu.CompilerParams(
            dimension_semantics=("parallel", "arbitrary"),
            vmem_limit_bytes=56 * 1024 * 1024,
        ),
        name="shared_tap_dilated_conv",
    )(xp, xp, kr, sc2, sh2)
    return tuple(o.reshape(B, H, W, C) for o in outs)
